# baseline (speedup 1.0000x reference)
import jax
import jax.numpy as jnp
from jax import lax
from jax.experimental import pallas as pl
from jax.experimental.pallas import tpu as pltpu
from jax.experimental.pallas import tpu_sc as plsc

NC, NS, LANES = 2, 16, 16
NW = NC * NS
NBUF = 4
WROW = 128
TR = 8


def kernel(input_seq, input_mask, table):
    L, B = input_seq.shape
    V, D = table.shape
    BPW = B // NW
    KD = D // LANES
    KB = BPW // LANES
    NWIN = V // WROW
    VPAD = (NWIN + 1) * WROW
    REM = NWIN % NW
    PER = NWIN // NW

    def body_a(tt_hbm, tail_hbm, out_hbm, w0, w1, o0, o1, r0, r1, s0, s1):
        win = (w0, w1)
        obuf = (o0, o1)
        rsem = (r0, r1)
        wsem = (s0, s1)

        sid = lax.axis_index("s")
        wid = sid * NC + lax.axis_index("c")
        cnt = PER + jnp.where(wid < REM, 1, 0)
        base_w = wid * PER + jnp.minimum(wid, REM)

        zero = jnp.zeros((LANES,), jnp.float32)

        def zb(j, c):
            for k in range(D // LANES, WROW // LANES):
                o0[j, pl.ds(k * LANES, LANES)] = zero
                o1[j, pl.ds(k * LANES, LANES)] = zero
            return c

        lax.fori_loop(0, WROW, zb, 0)

        def issue_reads(i, p):
            v0 = (base_w + i) * WROW
            for r in range(D // TR):
                pltpu.async_copy(
                    tt_hbm.at[pl.ds(TR * r, TR), pl.ds(v0, WROW)],
                    win[p].at[pl.ds(TR * r, TR)], rsem[p])

        def drain_reads(i, p):
            v0 = (base_w + i) * WROW
            for r in range(D // TR):
                pltpu.make_async_copy(
                    tt_hbm.at[pl.ds(TR * r, TR), pl.ds(v0, WROW)],
                    win[p].at[pl.ds(TR * r, TR)], rsem[p]).wait()

        def shuffle(p):
            @plsc.parallel_loop(0, WROW, unroll=8)
            def _(j):
                cols = jnp.full((LANES,), j, jnp.int32)
                for k in range(D // LANES):
                    rows = jnp.arange(k * LANES, (k + 1) * LANES,
                                      dtype=jnp.int32)
                    x = plsc.load_gather(win[p], [rows, cols])
                    obuf[p][j, pl.ds(k * LANES, LANES)] = x

        def write_out(i, p):
            v0 = (base_w + i) * WROW
            pltpu.async_copy(obuf[p], out_hbm.at[pl.ds(v0, WROW)], wsem[p])

        def drain_write(i, p):
            v0 = (base_w + i) * WROW
            pltpu.make_async_copy(obuf[p], out_hbm.at[pl.ds(v0, WROW)],
                                  wsem[p]).wait()

        for p in range(2):
            issue_reads(p, p)

        def gbody(g, c):
            for p in range(2):
                i = 2 * g + p

                @pl.when(i < cnt)
                def _():
                    drain_reads(i, p)

                    @pl.when(i >= 2)
                    def _():
                        drain_write(i - 2, p)

                    shuffle(p)
                    write_out(i, p)

                    @pl.when(i + 2 < cnt)
                    def _():
                        issue_reads(i + 2, p)

            return c

        lax.fori_loop(0, (PER + 2) // 2 + 1, gbody, 0)
        for p in range(2):
            pltpu.make_async_copy(obuf[p], out_hbm.at[pl.ds(0, WROW)],
                                  wsem[p]).wait()

        @pl.when(wid == NW - 1)
        def _():
            for r in range(D // TR):
                pltpu.sync_copy(tail_hbm.at[pl.ds(TR * r, TR)],
                                w0.at[pl.ds(TR * r, TR)])
            shuffle(0)
            pltpu.sync_copy(o0, out_hbm.at[pl.ds(NWIN * WROW, WROW)])

    def body_b(seq_hbm, mask_hbm, t2_hbm, out_hbm,
               seq_v, tgt_v, gb0, gb1, gb2, gb3, acc_v, cnt_v, shacc,
               gs0, gs1, gs2, gs3, ss0, ss1, ss2, ss3):
        gb = (gb0, gb1, gb2, gb3)
        gsem = (gs0, gs1, gs2, gs3)
        ssem = (ss0, ss1, ss2, ss3)

        sid = lax.axis_index("s")
        wid = sid * NC + lax.axis_index("c")
        base = wid * BPW
        srow = sid * (BPW + 1)

        pltpu.sync_copy(seq_hbm.at[:, pl.ds(base, BPW)], seq_v)
        pltpu.sync_copy(mask_hbm.at[:, pl.ds(base, BPW)], tgt_v)

        zero = jnp.zeros((LANES,), jnp.float32)

        def zbody(i, c):
            for k in range(KD):
                acc_v[i, pl.ds(k * LANES, LANES)] = zero
            return c

        lax.fori_loop(0, BPW + 1, zbody, 0)
        pltpu.sync_copy(acc_v, shacc.at[pl.ds(srow, BPW + 1)])

        iotas = [jnp.arange(k * LANES, (k + 1) * LANES, dtype=jnp.int32)
                 for k in range(KB)]
        trash = jnp.full((LANES,), BPW, jnp.int32)

        def cbody(l, cnts):
            out = []
            for k in range(KB):
                sl = pl.ds(k * LANES, LANES)
                m = tgt_v[l, sl]
                tgt_v[l, sl] = srow + jnp.where(m != 0, iotas[k], trash)
                seq_v[l, sl] = seq_v[l, sl] * 2
                out.append(cnts[k] + m)
            return tuple(out)

        cnts = lax.fori_loop(
            0, L, cbody,
            tuple(jnp.zeros((LANES,), jnp.int32) for _ in range(KB)))
        for k in range(KB):
            cnt_v[pl.ds(k * LANES, LANES)] = 1.0 / cnts[k].astype(jnp.float32)

        for b in range(NBUF):
            pltpu.async_copy(t2_hbm.at[seq_v.at[b]], gb[b], gsem[b])

        def step(l, b, issue_next):
            pltpu.make_async_copy(t2_hbm.at[seq_v.at[l]], gb[b],
                                  gsem[b]).wait()
            pltpu.async_copy(gb[b], shacc.at[tgt_v.at[l]], ssem[b], add=True)
            pltpu.make_async_copy(gb[b], shacc.at[tgt_v.at[l]],
                                  ssem[b]).wait()
            if issue_next:
                pltpu.async_copy(t2_hbm.at[seq_v.at[l + NBUF]], gb[b],
                                 gsem[b])

        NG = L // NBUF

        def gbody(g, c):
            for b in range(NBUF):
                step(g * NBUF + b, b, True)
            return c

        lax.fori_loop(0, NG - 1, gbody, 0)
        for b in range(NBUF):
            step((NG - 1) * NBUF + b, b, False)

        pltpu.sync_copy(shacc.at[pl.ds(srow, BPW)], acc_v.at[pl.ds(0, BPW)])

        lane_iota = jnp.arange(LANES, dtype=jnp.int32)

        def dbody(i, c):
            grp = i // LANES
            lane = i - grp * LANES
            rv = cnt_v[pl.ds(grp * LANES, LANES)]
            w = jnp.sum(jnp.where(lane_iota == lane, rv, 0.0))
            wv = jnp.full((LANES,), w, jnp.float32)
            for k in range(KD):
                sl = pl.ds(k * LANES, LANES)
                acc_v[i, sl] = acc_v[i, sl] * wv
            return c

        lax.fori_loop(0, BPW, dbody, 0)

        pltpu.sync_copy(acc_v.at[pl.ds(0, BPW)], out_hbm.at[pl.ds(base, BPW)])

    mesh = plsc.VectorSubcoreMesh(core_axis_name="c", subcore_axis_name="s",
                                  num_cores=NC, num_subcores=NS)

    table_t = jnp.transpose(table)
    tail128 = jnp.pad(table_t[:, NWIN * WROW:],
                      ((0, 0), (0, WROW - (V - NWIN * WROW))))

    run_a = pl.kernel(
        body_a,
        out_type=jax.ShapeDtypeStruct((VPAD, WROW), jnp.float32),
        mesh=mesh,
        compiler_params=pltpu.CompilerParams(needs_layout_passes=False,
                                             use_tc_tiling_on_sc=True,
                                             disable_bounds_checks=True),
        scratch_types=[
            pltpu.VMEM((D, WROW), jnp.float32),
            pltpu.VMEM((D, WROW), jnp.float32),
            pltpu.VMEM((WROW, WROW), jnp.float32),
            pltpu.VMEM((WROW, WROW), jnp.float32),
            *[pltpu.SemaphoreType.DMA for _ in range(4)],
        ],
    )
    t128 = run_a(table_t, tail128)
    t2 = jnp.reshape(t128, (2 * VPAD, D))

    run_b = pl.kernel(
        body_b,
        out_type=jax.ShapeDtypeStruct((B, D), jnp.float32),
        mesh=mesh,
        compiler_params=pltpu.CompilerParams(needs_layout_passes=False,
                                             use_tc_tiling_on_sc=False,
                                             disable_bounds_checks=True),
        scratch_types=[
            pltpu.VMEM((L, BPW), jnp.int32),
            pltpu.VMEM((L, BPW), jnp.int32),
            *[pltpu.VMEM((BPW, D), jnp.float32) for _ in range(NBUF)],
            pltpu.VMEM((BPW + 1, D), jnp.float32),
            pltpu.VMEM((BPW,), jnp.float32),
            pltpu.VMEM_SHARED((NS * (BPW + 1), D), jnp.float32),
            *[pltpu.SemaphoreType.DMA for _ in range(2 * NBUF)],
        ],
    )
    return run_b(input_seq, input_mask, t2)


# speedup vs baseline: 1.4185x; 1.4185x over previous
import jax
import jax.numpy as jnp
from jax import lax
from jax.experimental import pallas as pl
from jax.experimental.pallas import tpu as pltpu
from jax.experimental.pallas import tpu_sc as plsc

NC, NS, LANES = 2, 16, 16
NW = NC * NS
NBUF = 4
WROW = 128
TR = 8


def kernel(input_seq, input_mask, table):
    L, B = input_seq.shape
    V, D = table.shape
    BPW = B // NW
    KD = D // LANES
    KB = BPW // LANES

    def body_b(seq_hbm, mask_hbm, t2_hbm, out_hbm,
               seq_v, tgt_v, gb0, gb1, gb2, gb3, acc_v, cnt_v, shacc,
               gs0, gs1, gs2, gs3, ss0, ss1, ss2, ss3):
        gb = (gb0, gb1, gb2, gb3)
        gsem = (gs0, gs1, gs2, gs3)
        ssem = (ss0, ss1, ss2, ss3)

        sid = lax.axis_index("s")
        wid = sid * NC + lax.axis_index("c")
        base = wid * BPW
        srow = sid * (BPW + 1)

        pltpu.sync_copy(seq_hbm.at[:, pl.ds(base, BPW)], seq_v)
        pltpu.sync_copy(mask_hbm.at[:, pl.ds(base, BPW)], tgt_v)

        zero = jnp.zeros((LANES,), jnp.float32)

        def zbody(i, c):
            for k in range(KD):
                acc_v[i, pl.ds(k * LANES, LANES)] = zero
            return c

        lax.fori_loop(0, BPW + 1, zbody, 0)
        pltpu.sync_copy(acc_v, shacc.at[pl.ds(srow, BPW + 1)])

        iotas = [jnp.arange(k * LANES, (k + 1) * LANES, dtype=jnp.int32)
                 for k in range(KB)]
        trash = jnp.full((LANES,), BPW, jnp.int32)

        def cbody(l, cnts):
            out = []
            for k in range(KB):
                sl = pl.ds(k * LANES, LANES)
                m = tgt_v[l, sl]
                tgt_v[l, sl] = srow + jnp.where(m != 0, iotas[k], trash)
                seq_v[l, sl] = seq_v[l, sl] * 2
                out.append(cnts[k] + m)
            return tuple(out)

        cnts = lax.fori_loop(
            0, L, cbody,
            tuple(jnp.zeros((LANES,), jnp.int32) for _ in range(KB)))
        for k in range(KB):
            cnt_v[pl.ds(k * LANES, LANES)] = 1.0 / cnts[k].astype(jnp.float32)

        for b in range(NBUF):
            pltpu.async_copy(t2_hbm.at[seq_v.at[b]], gb[b], gsem[b])

        def step(l, b, issue_next):
            pltpu.make_async_copy(t2_hbm.at[seq_v.at[l]], gb[b],
                                  gsem[b]).wait()
            pltpu.async_copy(gb[b], shacc.at[tgt_v.at[l]], ssem[b], add=True)
            pltpu.make_async_copy(gb[b], shacc.at[tgt_v.at[l]],
                                  ssem[b]).wait()
            if issue_next:
                pltpu.async_copy(t2_hbm.at[seq_v.at[l + NBUF]], gb[b],
                                 gsem[b])

        NG = L // NBUF

        def gbody(g, c):
            for b in range(NBUF):
                step(g * NBUF + b, b, True)
            return c

        lax.fori_loop(0, NG - 1, gbody, 0)
        for b in range(NBUF):
            step((NG - 1) * NBUF + b, b, False)

        pltpu.sync_copy(shacc.at[pl.ds(srow, BPW)], acc_v.at[pl.ds(0, BPW)])

        lane_iota = jnp.arange(LANES, dtype=jnp.int32)

        def dbody(i, c):
            grp = i // LANES
            lane = i - grp * LANES
            rv = cnt_v[pl.ds(grp * LANES, LANES)]
            w = jnp.sum(jnp.where(lane_iota == lane, rv, 0.0))
            wv = jnp.full((LANES,), w, jnp.float32)
            for k in range(KD):
                sl = pl.ds(k * LANES, LANES)
                acc_v[i, sl] = acc_v[i, sl] * wv
            return c

        lax.fori_loop(0, BPW, dbody, 0)

        pltpu.sync_copy(acc_v.at[pl.ds(0, BPW)], out_hbm.at[pl.ds(base, BPW)])

    mesh = plsc.VectorSubcoreMesh(core_axis_name="c", subcore_axis_name="s",
                                  num_cores=NC, num_subcores=NS)

    t128 = jnp.pad(table, ((0, 0), (0, WROW - D)))
    t2 = jnp.reshape(t128, (2 * V, D))

    run_b = pl.kernel(
        body_b,
        out_type=jax.ShapeDtypeStruct((B, D), jnp.float32),
        mesh=mesh,
        compiler_params=pltpu.CompilerParams(needs_layout_passes=False,
                                             use_tc_tiling_on_sc=False,
                                             disable_bounds_checks=True),
        scratch_types=[
            pltpu.VMEM((L, BPW), jnp.int32),
            pltpu.VMEM((L, BPW), jnp.int32),
            *[pltpu.VMEM((BPW, D), jnp.float32) for _ in range(NBUF)],
            pltpu.VMEM((BPW + 1, D), jnp.float32),
            pltpu.VMEM((BPW,), jnp.float32),
            pltpu.VMEM_SHARED((NS * (BPW + 1), D), jnp.float32),
            *[pltpu.SemaphoreType.DMA for _ in range(2 * NBUF)],
        ],
    )
    return run_b(input_seq, input_mask, t2)
